# baseline (speedup 1.0000x reference)
import jax
import jax.numpy as jnp
from jax.experimental import pallas as pl
from jax.experimental.pallas import tpu as pltpu


def _copy_body(in_hbm, out_hbm, sem):
    pltpu.make_async_copy(in_hbm, out_hbm, sem).start()
    pltpu.make_async_copy(in_hbm, out_hbm, sem).wait()


def kernel(inputs, transition_params):
    del transition_params
    x = inputs.astype(jnp.float32)
    return pl.pallas_call(
        _copy_body,
        out_shape=jax.ShapeDtypeStruct(x.shape, jnp.float32),
        in_specs=[pl.BlockSpec(memory_space=pl.ANY)],
        out_specs=pl.BlockSpec(memory_space=pl.ANY),
        scratch_shapes=[pltpu.SemaphoreType.DMA],
    )(x)


# speedup vs baseline: 15.1318x; 15.1318x over previous
import jax
import jax.numpy as jnp
from jax.experimental import pallas as pl
from jax.experimental.pallas import tpu as pltpu


def _copy_body(in_ref, out_ref):
    out_ref[...] = in_ref[...]


def kernel(inputs, transition_params):
    del transition_params
    x = inputs.astype(jnp.float32)
    b, s, c = x.shape
    blk = 4
    return pl.pallas_call(
        _copy_body,
        out_shape=jax.ShapeDtypeStruct((b, s, c), jnp.float32),
        grid=(b // blk,),
        in_specs=[pl.BlockSpec((blk, s, c), lambda i: (i, 0, 0))],
        out_specs=pl.BlockSpec((blk, s, c), lambda i: (i, 0, 0)),
    )(x)
